# baseline (speedup 1.0000x reference)
import functools
import jax
import jax.numpy as jnp
from jax import lax
from jax.experimental import pallas as pl
from jax.experimental.pallas import tpu as pltpu
from jax.experimental.pallas import tpu_sc as plsc

_N = 100000
_K = 10000
_F = 128
_G = 80
_NG = _N // _G
_NW = 32
_GPW = 40
_NB = 4
_PF = 3
_PEEL_LO = _NB - _PF
_PEEL_HI = _PF
_STEADY = _GPW - _PEEL_LO - _PEEL_HI
_OUTER = _STEADY // _NB

_mesh = plsc.VectorSubcoreMesh(core_axis_name="c", subcore_axis_name="s")


@functools.partial(
    pl.kernel,
    mesh=_mesh,
    out_type=jax.ShapeDtypeStruct((_NG, _G, _F), jnp.float32),
    scratch_types=[
        pltpu.VMEM_SHARED((_K, _F), jnp.float32),
        pltpu.VMEM((_GPW * _G,), jnp.int32),
        pltpu.VMEM((_GPW * _G,), jnp.float32),
        pltpu.VMEM((_NB, _G, _F), jnp.float32),
        pltpu.SemaphoreType.DMA,
        pltpu.SemaphoreType.DMA,
        pltpu.SemaphoreType.DMA,
        pltpu.SemaphoreType.DMA,
        pltpu.SemaphoreType.DMA,
        pltpu.SemaphoreType.DMA,
        pltpu.SemaphoreType.DMA,
        pltpu.SemaphoreType.DMA,
        pltpu.SemaphoreType.DMA,
    ],
)
def _lift(x_hbm, idx_hbm, s_hbm, out_hbm, x_sh, idx_sl, s_sl, rows_v,
          sg0, sg1, sg2, sg3, sw0, sw1, sw2, sw3, sst):
    sems_g = [sg0, sg1, sg2, sg3]
    sems_w = [sw0, sw1, sw2, sw3]
    sid = lax.axis_index("s")
    wid = sid * 2 + lax.axis_index("c")
    base = jnp.minimum(wid * _GPW, _NG - _GPW)

    _C = 632
    off = jnp.minimum(sid * (_C // 8), (_K - _C) // 8) * 8
    stage_cp = pltpu.async_copy(
        x_hbm.at[pl.ds(off, _C)], x_sh.at[pl.ds(off, _C)], sst)

    pltpu.sync_copy(idx_hbm.at[pl.ds(base * _G, _GPW * _G)], idx_sl)
    pltpu.sync_copy(s_hbm.at[pl.ds(base * _G, _GPW * _G)], s_sl)

    def fire_gather(g, slot, src):
        pltpu.async_copy(
            src.at[idx_sl.at[pl.ds(g * _G, _G)]],
            rows_v.at[slot], sems_g[slot])

    def wait_gather(g, slot, src):
        pltpu.make_async_copy(
            src.at[idx_sl.at[pl.ds(g * _G, _G)]],
            rows_v.at[slot], sems_g[slot]).wait()

    def fire_write(g, slot):
        pltpu.async_copy(rows_v.at[slot], out_hbm.at[base + g], sems_w[slot])

    def wait_write(slot):
        pltpu.make_async_copy(
            rows_v.at[slot], out_hbm.at[0], sems_w[slot]).wait()

    def scale(g, slot):
        buf = rows_v.at[slot]

        def blk_body(bb, c):
            s16 = s_sl[pl.ds(g * _G + bb * 16, 16)]
            for r in range(16):
                s = s16[r]
                i = bb * 16 + r
                for j in range(_F // 16):
                    sl = pl.ds(j * 16, 16)
                    buf[i, sl] = buf[i, sl] * s
            return c

        lax.fori_loop(0, _G // 16, blk_body, 0)

    for b in range(_NB):
        fire_gather(b, b, x_hbm)

    for g0 in range(_PEEL_LO):
        wait_gather(g0, g0, x_hbm)
        scale(g0, g0)
        fire_write(g0, g0)

    stage_cp.wait()
    plsc.subcore_barrier()

    def outer_body(t0, carry):
        for b in range(_NB):
            g = _PEEL_LO + t0 * _NB + b
            slot = (_PEEL_LO + b) % _NB
            bh = b
            wait_write(bh)
            fire_gather(g + _PF, bh, x_sh)
            wait_gather(g, slot, x_sh)
            scale(g, slot)
            fire_write(g, slot)
        return carry

    lax.fori_loop(0, _OUTER, outer_body, 0)

    for k in range(_PEEL_HI):
        g = _GPW - _PEEL_HI + k
        slot = g % _NB
        wait_write((g + _PF) % _NB)
        wait_gather(g, slot, x_sh)
        scale(g, slot)
        fire_write(g, slot)

    for k in range(_NB - _PF):
        wait_write((_GPW - (_NB - _PF) + k) % _NB)


def kernel(x_pool, cluster, s_val):
    idx = cluster.astype(jnp.int32)
    out = _lift(x_pool, idx, s_val)
    return out.reshape(_N, _F)


# speedup vs baseline: 1.0841x; 1.0841x over previous
import functools
import jax
import jax.numpy as jnp
from jax import lax
from jax.experimental import pallas as pl
from jax.experimental.pallas import tpu as pltpu
from jax.experimental.pallas import tpu_sc as plsc

_N = 100000
_K = 10000
_F = 128
_G = 80
_NG = _N // _G
_GPW = 40
_NB = 4
_PF = 2
_PEEL_LO = _NB - _PF
_PEEL_B = 4
_PEEL_HI = _PF
_STEADY = _GPW - _PEEL_LO - _PEEL_B - _PEEL_HI
_OUTER = _STEADY // _NB

_mesh = plsc.VectorSubcoreMesh(core_axis_name="c", subcore_axis_name="s")


@functools.partial(
    pl.kernel,
    mesh=_mesh,
    out_type=jax.ShapeDtypeStruct((_NG, _G, _F), jnp.float32),
    scratch_types=[
        pltpu.VMEM_SHARED((_K, _F), jnp.float32),
        pltpu.VMEM((_GPW * _G,), jnp.int32),
        pltpu.VMEM((_GPW * _G,), jnp.float32),
        pltpu.VMEM((_NB, _G, _F), jnp.float32),
        pltpu.SemaphoreType.DMA,
        pltpu.SemaphoreType.DMA,
        pltpu.SemaphoreType.DMA,
        pltpu.SemaphoreType.DMA,
        pltpu.SemaphoreType.DMA,
        pltpu.SemaphoreType.DMA,
        pltpu.SemaphoreType.DMA,
        pltpu.SemaphoreType.DMA,
        pltpu.SemaphoreType.DMA,
    ],
)
def _lift(x_hbm, idx_hbm, s_hbm, out_hbm, x_sh, idx_sl, s_sl, rows_v,
          sg0, sg1, sg2, sg3, sw0, sw1, sw2, sw3, sst):
    sems_g = [sg0, sg1, sg2, sg3]
    sems_w = [sw0, sw1, sw2, sw3]
    sid = lax.axis_index("s")
    wid = sid * 2 + lax.axis_index("c")
    base = jnp.minimum(wid * _GPW, _NG - _GPW)

    _C = 632
    off = jnp.minimum(sid * (_C // 8), (_K - _C) // 8) * 8
    stage_cp = pltpu.async_copy(
        x_hbm.at[pl.ds(off, _C)], x_sh.at[pl.ds(off, _C)], sst)

    pltpu.sync_copy(idx_hbm.at[pl.ds(base * _G, _GPW * _G)], idx_sl)
    pltpu.sync_copy(s_hbm.at[pl.ds(base * _G, _GPW * _G)], s_sl)

    def fire_gather(g, slot, src):
        pltpu.async_copy(
            src.at[idx_sl.at[pl.ds(g * _G, _G)]],
            rows_v.at[slot], sems_g[slot])

    def wait_gather(g, slot, src):
        pltpu.make_async_copy(
            src.at[idx_sl.at[pl.ds(g * _G, _G)]],
            rows_v.at[slot], sems_g[slot]).wait()

    def fire_write(g, slot):
        pltpu.async_copy(rows_v.at[slot], out_hbm.at[base + g], sems_w[slot])

    def wait_write(slot):
        pltpu.make_async_copy(
            rows_v.at[slot], out_hbm.at[0], sems_w[slot]).wait()

    def scale(g, slot):
        buf = rows_v.at[slot]

        def blk_body(bb, c):
            s16 = s_sl[pl.ds(g * _G + bb * 16, 16)]
            for r in range(16):
                s = s16[r]
                i = bb * 16 + r
                for j in range(_F // 16):
                    sl = pl.ds(j * 16, 16)
                    buf[i, sl] = buf[i, sl] * s
            return c

        lax.fori_loop(0, _G // 16, blk_body, 0)

    for b in range(_NB):
        fire_gather(b, b, x_hbm)

    for g0 in range(_PEEL_LO):
        wait_gather(g0, g0, x_hbm)
        scale(g0, g0)
        fire_write(g0, g0)

    for g in range(_PEEL_LO, _PEEL_LO + _PEEL_B):
        wait_write((g + _PF) % _NB)
        fire_gather(g + _PF, (g + _PF) % _NB, x_hbm)
        wait_gather(g, g % _NB, x_hbm)
        scale(g, g % _NB)
        fire_write(g, g % _NB)

    stage_cp.wait()
    plsc.subcore_barrier()

    def outer_body(t0, carry):
        for b in range(_NB):
            g = _PEEL_LO + _PEEL_B + t0 * _NB + b
            slot = (_PEEL_LO + _PEEL_B + b) % _NB
            bh = b
            wait_write(bh)
            fire_gather(g + _PF, bh, x_sh)
            wait_gather(g, slot, x_sh)
            scale(g, slot)
            fire_write(g, slot)
        return carry

    lax.fori_loop(0, _OUTER, outer_body, 0)

    for k in range(_PEEL_HI):
        g = _GPW - _PEEL_HI + k
        slot = g % _NB
        wait_write((g + _PF) % _NB)
        wait_gather(g, slot, x_sh)
        scale(g, slot)
        fire_write(g, slot)

    for k in range(_NB - _PF):
        wait_write((_GPW - (_NB - _PF) + k) % _NB)


def kernel(x_pool, cluster, s_val):
    idx = cluster.astype(jnp.int32)
    out = _lift(x_pool, idx, s_val)
    return out.reshape(_N, _F)


# speedup vs baseline: 1.1510x; 1.0617x over previous
import functools
import jax
import jax.numpy as jnp
from jax import lax
from jax.experimental import pallas as pl
from jax.experimental.pallas import tpu as pltpu
from jax.experimental.pallas import tpu_sc as plsc

_N = 100000
_K = 10000
_F = 128
_G = 80
_NG = _N // _G
_GPW = 40
_NB = 4
_PF = 2
_PEEL_LO = _NB - _PF
_PEEL_HI = _PF
_STEADY = _GPW - _PEEL_LO - _PEEL_HI
_OUTER = _STEADY // _NB

_mesh = plsc.VectorSubcoreMesh(core_axis_name="c", subcore_axis_name="s")


@functools.partial(
    pl.kernel,
    mesh=_mesh,
    out_type=jax.ShapeDtypeStruct((_NG, _G, _F), jnp.float32),
    scratch_types=[
        pltpu.VMEM_SHARED((_K, _F), jnp.float32),
        pltpu.VMEM((_GPW * _G,), jnp.int32),
        pltpu.VMEM((_GPW * _G,), jnp.float32),
        pltpu.VMEM((_NB, _G, _F), jnp.float32),
        pltpu.SemaphoreType.DMA,
        pltpu.SemaphoreType.DMA,
        pltpu.SemaphoreType.DMA,
        pltpu.SemaphoreType.DMA,
        pltpu.SemaphoreType.DMA,
        pltpu.SemaphoreType.DMA,
        pltpu.SemaphoreType.DMA,
        pltpu.SemaphoreType.DMA,
        pltpu.SemaphoreType.DMA,
    ],
)
def _lift(x_hbm, idx_hbm, s_hbm, out_hbm, x_sh, idx_sl, s_sl, rows_v,
          sg0, sg1, sg2, sg3, sw0, sw1, sw2, sw3, sst):
    sems_g = [sg0, sg1, sg2, sg3]
    sems_w = [sw0, sw1, sw2, sw3]
    sid = lax.axis_index("s")
    wid = sid * 2 + lax.axis_index("c")
    base = jnp.minimum(wid * _GPW, _NG - _GPW)

    _C = 632
    off = jnp.minimum(sid * (_C // 8), (_K - _C) // 8) * 8
    stage_cp = pltpu.async_copy(
        x_hbm.at[pl.ds(off, _C)], x_sh.at[pl.ds(off, _C)], sst)

    pltpu.sync_copy(idx_hbm.at[pl.ds(base * _G, _GPW * _G)], idx_sl)
    pltpu.sync_copy(s_hbm.at[pl.ds(base * _G, _GPW * _G)], s_sl)

    def fire_gather(g, slot, src):
        pltpu.async_copy(
            src.at[idx_sl.at[pl.ds(g * _G, _G)]],
            rows_v.at[slot], sems_g[slot])

    def wait_gather(g, slot, src):
        pltpu.make_async_copy(
            src.at[idx_sl.at[pl.ds(g * _G, _G)]],
            rows_v.at[slot], sems_g[slot]).wait()

    def fire_write(g, slot):
        pltpu.async_copy(rows_v.at[slot], out_hbm.at[base + g], sems_w[slot])

    def wait_write(slot):
        pltpu.make_async_copy(
            rows_v.at[slot], out_hbm.at[0], sems_w[slot]).wait()

    def scale(g, slot):
        buf = rows_v.at[slot]

        def blk_body(bb, c):
            s16 = s_sl[pl.ds(g * _G + bb * 16, 16)]
            for r in range(16):
                s = s16[r]
                i = bb * 16 + r
                for j in range(_F // 16):
                    sl = pl.ds(j * 16, 16)
                    buf[i, sl] = buf[i, sl] * s
            return c

        lax.fori_loop(0, _G // 16, blk_body, 0)

    for b in range(_NB):
        fire_gather(b, b, x_hbm)

    for g0 in range(_PEEL_LO):
        wait_gather(g0, g0, x_hbm)
        scale(g0, g0)
        fire_write(g0, g0)

    stage_cp.wait()
    plsc.subcore_barrier()

    def outer_body(t0, carry):
        for b in range(_NB):
            g = _PEEL_LO + t0 * _NB + b
            slot = (_PEEL_LO + b) % _NB
            bh = b
            wait_write(bh)
            fire_gather(g + _PF, bh, x_sh)
            wait_gather(g, slot, x_sh)
            scale(g, slot)
            fire_write(g, slot)
        return carry

    lax.fori_loop(0, _OUTER, outer_body, 0)

    for k in range(_PEEL_HI):
        g = _GPW - _PEEL_HI + k
        slot = g % _NB
        wait_write((g + _PF) % _NB)
        wait_gather(g, slot, x_sh)
        scale(g, slot)
        fire_write(g, slot)

    for k in range(_NB - _PF):
        wait_write((_GPW - (_NB - _PF) + k) % _NB)


def kernel(x_pool, cluster, s_val):
    idx = cluster.astype(jnp.int32)
    out = _lift(x_pool, idx, s_val)
    return out.reshape(_N, _F)
